# baseline (speedup 1.0000x reference)
import functools
import math

import jax
import jax.numpy as jnp
from jax import lax
from jax.experimental import pallas as pl
from jax.experimental.pallas import tpu as pltpu

_SW = 32
_FB = 16
_EMB = 256
_TOK = 512
_G = _TOK * _FB // 128
_BLK = 128


def _sinusoidal(L, D):
  pos = jnp.arange(L, dtype=jnp.float32)[:, None]
  idx = jnp.arange(D, dtype=jnp.float32)[None, :]
  angle = pos / jnp.power(10000.0, 2.0 * jnp.floor(idx / 2.0) / D)
  return jnp.where(jnp.arange(D)[None, :] % 2 == 0, jnp.sin(angle),
                   jnp.cos(angle))


def _stage_a(xt_ref, wq_ref, wk_ref, wv_ref, w1_ref, w2_ref,
             wout_ref, bq_ref, bk_ref, bv_ref, b1_ref, b2_ref,
             bout_ref, emb_ref, s1_ref, s2_ref):
  step = pl.program_id(0)
  x3 = xt_ref[...]
  def mm(a, w):
    return lax.dot_general(a, w, (((2,), (0,)), ((), ())),
                           preferred_element_type=jnp.float32)
  def mmt(a, w):
    return lax.dot_general(a, w, (((1,), (0,)), ((), ())),
                           preferred_element_type=jnp.float32)
  q = mmt(x3, wq_ref[...]).reshape(_G, 128, _SW) + bq_ref[...][None]
  k = mmt(x3, wk_ref[...]).reshape(_G, 128, _SW) + bk_ref[...][None]
  v = mmt(x3, wv_ref[...]).reshape(_G, 128, _SW) + bv_ref[...][None]

  s = lax.dot_general(q, k, (((2,), (2,)), ((0,), (0,))),
                      preferred_element_type=jnp.float32)
  i = lax.broadcasted_iota(jnp.int32, (128, 128), 0)
  j = lax.broadcasted_iota(jnp.int32, (128, 128), 1)
  mask = jnp.where((i // _FB) == (j // _FB), 0.0, -1e30).astype(jnp.float32)
  s = s + mask
  m = jnp.max(s, axis=-1, keepdims=True)
  e = jnp.exp(s - m)
  o = lax.dot_general(e, v, (((2,), (1,)), ((0,), (0,))),
                      preferred_element_type=jnp.float32)
  o = o / jnp.sum(e, axis=-1, keepdims=True)
  h = jnp.maximum(mm(o, w1_ref[...]) + b1_ref[...][None], 0.0)
  e2 = mm(h, w2_ref[...]) + b2_ref[...][None]
  e3 = e2.reshape(_TOK, _FB, _SW)
  ecat = jnp.concatenate([e3[:, f, :] for f in range(_FB)], axis=1)
  emb = ecat @ wout_ref[...] + bout_ref[...]
  emb_ref[...] = emb
  ssum = jnp.sum(emb).reshape(1, 1)
  ssq = jnp.sum(emb * emb).reshape(1, 1)

  @pl.when(step == 0)
  def _():
    s1_ref[...] = ssum
    s2_ref[...] = ssq

  @pl.when(step != 0)
  def _():
    s1_ref[...] += ssum
    s2_ref[...] += ssq


def _stage_b(emb_ref, r_ref, oc_ref, pe_ref, mu_ref, inv_ref, out_ref):
  T = emb_ref.shape[1]
  nblk = T // _BLK
  mu = mu_ref[0, 0]
  inv = inv_ref[0, 0]

  ti = lax.broadcasted_iota(jnp.int32, (_BLK, _BLK), 0)
  tj = lax.broadcasted_iota(jnp.int32, (_BLK, _BLK), 1)
  tri_low = jnp.where(tj <= ti, 1.0, 0.0).astype(jnp.float32)
  upper = (tj >= ti)

  def body(it, carry):
    jb = nblk - 1 - it
    cs, cc = carry
    ob = oc_ref[0, pl.ds(jb * _BLK, _BLK), :]
    cum = tri_low @ ob
    same = cum == cum.reshape(1, _BLK)
    msk = jnp.where(same & upper, 1.0, 0.0).astype(jnp.float32)
    eb = (emb_ref[0, pl.ds(jb * _BLK, _BLK), :] - mu) * inv
    sblk = msk @ eb
    cblk = jnp.sum(msk, axis=1, keepdims=True)
    nb = jnp.where(jb < nblk - 1, jb + 1, nblk - 1)
    onext = oc_ref[0, pl.ds(nb * _BLK, 1), :]
    cont = jnp.where((jb < nblk - 1) & (onext[0, 0] == 0.0), 1.0, 0.0)
    trail = jnp.where(cum == cum[_BLK - 1, 0], cont, 0.0)
    sblk = sblk + trail * cs
    cblk = cblk + trail * cc
    gidx = jb * _BLK + lax.broadcasted_iota(jnp.int32, (_BLK, 1), 0)
    first = (ob > 0.0) | (gidx == 0)
    bm = jnp.where(first, sblk / cblk, 0.0)
    out_ref[0, pl.ds(jb * _BLK, _BLK), :] = (
        r_ref[0, pl.ds(jb * _BLK, _BLK), :]
        + pe_ref[pl.ds(jb * _BLK, _BLK), :] + eb + bm)
    return (sblk[0:1, :], cblk[0:1, :])

  lax.fori_loop(0, nblk, body,
                (jnp.zeros((1, _EMB), jnp.float32),
                 jnp.zeros((1, 1), jnp.float32)))


@jax.jit
def kernel(x, o_enc, r_enc, Wq, bq, Wk, bk, Wv, bv, Wo, bo, W1, b1, W2, b2,
           Wout, bout):
  B, T, F = x.shape
  B2 = B * T
  xt = x.reshape(B2, _SW, _FB)
  pe_eff = _sinusoidal(_FB, _SW) * 0.01 + 0.01
  rs = 1.0 / math.sqrt(_SW)
  Wqs = Wq * rs
  Bq = jnp.tile((pe_eff @ Wq + bq) * rs, (128 // _FB, 1))
  Bk = jnp.tile(pe_eff @ Wk + bk, (128 // _FB, 1))
  Bv = jnp.tile(pe_eff @ Wv + bv, (128 // _FB, 1))
  WoW1 = Wo @ W1
  b1f = (bo @ W1 + b1).reshape(1, _SW * 4)
  Wout3 = Wout.reshape(_SW, _FB, _EMB).transpose(1, 0, 2).reshape(F, _EMB)

  grid_a = B2 // _TOK
  emb, s1, s2 = pl.pallas_call(
      _stage_a,
      grid=(grid_a,),
      in_specs=[
          pl.BlockSpec((_TOK, _SW, _FB), lambda i: (i, 0, 0)),
          *[pl.BlockSpec(w.shape, lambda i: (0,) * w.ndim) for w in
            (Wqs, Wk, Wv, WoW1, W2)],
          pl.BlockSpec((F, _EMB), lambda i: (0, 0)),
          pl.BlockSpec((128, _SW), lambda i: (0, 0)),
          pl.BlockSpec((128, _SW), lambda i: (0, 0)),
          pl.BlockSpec((128, _SW), lambda i: (0, 0)),
          pl.BlockSpec((1, _SW * 4), lambda i: (0, 0)),
          pl.BlockSpec((1, _SW), lambda i: (0, 0)),
          pl.BlockSpec((1, _EMB), lambda i: (0, 0)),
      ],
      out_specs=[
          pl.BlockSpec((_TOK, _EMB), lambda i: (i, 0)),
          pl.BlockSpec((1, 1), lambda i: (0, 0)),
          pl.BlockSpec((1, 1), lambda i: (0, 0)),
      ],
      out_shape=[
          jax.ShapeDtypeStruct((B2, _EMB), jnp.float32),
          jax.ShapeDtypeStruct((1, 1), jnp.float32),
          jax.ShapeDtypeStruct((1, 1), jnp.float32),
      ],
  )(xt, Wqs, Wk, Wv, WoW1, W2, Wout3,
    Bq, Bk, Bv, b1f, b2.reshape(1, _SW), bout.reshape(1, _EMB))

  n = jnp.float32(B2 * _EMB)
  mu = s1[0, 0] / n
  var = s2[0, 0] / n - mu * mu
  inv = 1.0 / jnp.sqrt(var + 1e-8)
  mu = mu.reshape(1, 1)
  inv = inv.reshape(1, 1)

  pe_big = _sinusoidal(T, _EMB)
  o_col = o_enc.astype(jnp.float32).reshape(B, T, 1)

  out = pl.pallas_call(
      _stage_b,
      grid=(B,),
      in_specs=[
          pl.BlockSpec((1, T, _EMB), lambda i: (i, 0, 0)),
          pl.BlockSpec((1, T, _EMB), lambda i: (i, 0, 0)),
          pl.BlockSpec((1, T, 1), lambda i: (i, 0, 0)),
          pl.BlockSpec((T, _EMB), lambda i: (0, 0)),
          pl.BlockSpec((1, 1), lambda i: (0, 0)),
          pl.BlockSpec((1, 1), lambda i: (0, 0)),
      ],
      out_specs=pl.BlockSpec((1, T, _EMB), lambda i: (i, 0, 0)),
      out_shape=jax.ShapeDtypeStruct((B, T, _EMB), jnp.float32),
  )(emb.reshape(B, T, _EMB), r_enc, o_col, pe_big, mu, inv)
  return out


# speedup vs baseline: 1.1196x; 1.1196x over previous
import functools
import math

import jax
import jax.numpy as jnp
from jax import lax
from jax.experimental import pallas as pl
from jax.experimental.pallas import tpu as pltpu

_SW = 32
_FB = 16
_EMB = 256
_TOK = 512
_G = _TOK * _FB // 128
_BLK = 128


def _sinusoidal(L, D):
  pos = jnp.arange(L, dtype=jnp.float32)[:, None]
  idx = jnp.arange(D, dtype=jnp.float32)[None, :]
  angle = pos / jnp.power(10000.0, 2.0 * jnp.floor(idx / 2.0) / D)
  return jnp.where(jnp.arange(D)[None, :] % 2 == 0, jnp.sin(angle),
                   jnp.cos(angle))


def _stage_a(xt_ref, wq_ref, wk_ref, wv_ref, w1_ref, w2_ref,
             wout_ref, bq_ref, bk_ref, bv_ref, b1_ref, b2_ref,
             bout_ref, emb_ref, s1_ref, s2_ref):
  step = pl.program_id(0)
  x3 = xt_ref[...]
  def mm(a, w):
    return lax.dot_general(a, w, (((2,), (0,)), ((), ())),
                           preferred_element_type=jnp.float32)
  q = mm(x3, wq_ref[...]) + bq_ref[...][None]
  k = mm(x3, wk_ref[...]) + bk_ref[...][None]
  v = mm(x3, wv_ref[...]) + bv_ref[...][None]

  s = lax.dot_general(q, k, (((2,), (2,)), ((0,), (0,))),
                      preferred_element_type=jnp.float32)
  i = lax.broadcasted_iota(jnp.int32, (128, 128), 0)
  j = lax.broadcasted_iota(jnp.int32, (128, 128), 1)
  mask = jnp.where((i // _FB) == (j // _FB), 0.0, -1e30).astype(jnp.float32)
  s = s + mask
  m = jnp.max(s, axis=-1, keepdims=True)
  e = jnp.exp(s - m)
  o = lax.dot_general(e, v, (((2,), (1,)), ((0,), (0,))),
                      preferred_element_type=jnp.float32)
  o = o / jnp.sum(e, axis=-1, keepdims=True)
  h = jnp.maximum(mm(o, w1_ref[...]) + b1_ref[...][None], 0.0)
  e2 = mm(h, w2_ref[...]) + b2_ref[...][None]
  e3 = e2.reshape(_TOK, _FB, _SW)
  ecat = jnp.concatenate([e3[:, f, :] for f in range(_FB)], axis=1)
  emb = ecat @ wout_ref[...] + bout_ref[...]
  emb_ref[...] = emb
  ssum = jnp.sum(emb).reshape(1, 1)
  ssq = jnp.sum(emb * emb).reshape(1, 1)

  @pl.when(step == 0)
  def _():
    s1_ref[...] = ssum
    s2_ref[...] = ssq

  @pl.when(step != 0)
  def _():
    s1_ref[...] += ssum
    s2_ref[...] += ssq


def _stage_b(emb_ref, r_ref, oc_ref, pe_ref, mu_ref, inv_ref, out_ref):
  T = emb_ref.shape[1]
  nblk = T // _BLK
  mu = mu_ref[0, 0]
  inv = inv_ref[0, 0]

  ti = lax.broadcasted_iota(jnp.int32, (_BLK, _BLK), 0)
  tj = lax.broadcasted_iota(jnp.int32, (_BLK, _BLK), 1)
  tri_low = jnp.where(tj <= ti, 1.0, 0.0).astype(jnp.float32)
  upper = (tj >= ti)

  def body(it, carry):
    jb = nblk - 1 - it
    cs, cc = carry
    ob = oc_ref[0, pl.ds(jb * _BLK, _BLK), :]
    cum = tri_low @ ob
    same = cum == cum.reshape(1, _BLK)
    msk = jnp.where(same & upper, 1.0, 0.0).astype(jnp.float32)
    eb = (emb_ref[0, pl.ds(jb * _BLK, _BLK), :] - mu) * inv
    sblk = msk @ eb
    cblk = jnp.sum(msk, axis=1, keepdims=True)
    nb = jnp.where(jb < nblk - 1, jb + 1, nblk - 1)
    onext = oc_ref[0, pl.ds(nb * _BLK, 1), :]
    cont = jnp.where((jb < nblk - 1) & (onext[0, 0] == 0.0), 1.0, 0.0)
    trail = jnp.where(cum == cum[_BLK - 1, 0], cont, 0.0)
    sblk = sblk + trail * cs
    cblk = cblk + trail * cc
    gidx = jb * _BLK + lax.broadcasted_iota(jnp.int32, (_BLK, 1), 0)
    first = (ob > 0.0) | (gidx == 0)
    bm = jnp.where(first, sblk / cblk, 0.0)
    out_ref[0, pl.ds(jb * _BLK, _BLK), :] = (
        r_ref[0, pl.ds(jb * _BLK, _BLK), :]
        + pe_ref[pl.ds(jb * _BLK, _BLK), :] + eb + bm)
    return (sblk[0:1, :], cblk[0:1, :])

  lax.fori_loop(0, nblk, body,
                (jnp.zeros((1, _EMB), jnp.float32),
                 jnp.zeros((1, 1), jnp.float32)))


@jax.jit
def kernel(x, o_enc, r_enc, Wq, bq, Wk, bk, Wv, bv, Wo, bo, W1, b1, W2, b2,
           Wout, bout):
  B, T, F = x.shape
  B2 = B * T
  xt = jnp.transpose(x.reshape(B2, _SW, _FB), (0, 2, 1)).reshape(
      B2 * _FB // 128, 128, _SW)
  pe_eff = _sinusoidal(_FB, _SW) * 0.01 + 0.01
  rs = 1.0 / math.sqrt(_SW)
  Wqs = Wq * rs
  Bq = jnp.tile((pe_eff @ Wq + bq) * rs, (128 // _FB, 1))
  Bk = jnp.tile(pe_eff @ Wk + bk, (128 // _FB, 1))
  Bv = jnp.tile(pe_eff @ Wv + bv, (128 // _FB, 1))
  WoW1 = Wo @ W1
  b1f = (bo @ W1 + b1).reshape(1, _SW * 4)
  Wout3 = Wout.reshape(_SW, _FB, _EMB).transpose(1, 0, 2).reshape(F, _EMB)

  grid_a = B2 // _TOK
  emb, s1, s2 = pl.pallas_call(
      _stage_a,
      grid=(grid_a,),
      in_specs=[
          pl.BlockSpec((_G, 128, _SW), lambda i: (i, 0, 0)),
          *[pl.BlockSpec(w.shape, lambda i: (0,) * w.ndim) for w in
            (Wqs, Wk, Wv, WoW1, W2)],
          pl.BlockSpec((F, _EMB), lambda i: (0, 0)),
          pl.BlockSpec((128, _SW), lambda i: (0, 0)),
          pl.BlockSpec((128, _SW), lambda i: (0, 0)),
          pl.BlockSpec((128, _SW), lambda i: (0, 0)),
          pl.BlockSpec((1, _SW * 4), lambda i: (0, 0)),
          pl.BlockSpec((1, _SW), lambda i: (0, 0)),
          pl.BlockSpec((1, _EMB), lambda i: (0, 0)),
      ],
      out_specs=[
          pl.BlockSpec((_TOK, _EMB), lambda i: (i, 0)),
          pl.BlockSpec((1, 1), lambda i: (0, 0)),
          pl.BlockSpec((1, 1), lambda i: (0, 0)),
      ],
      out_shape=[
          jax.ShapeDtypeStruct((B2, _EMB), jnp.float32),
          jax.ShapeDtypeStruct((1, 1), jnp.float32),
          jax.ShapeDtypeStruct((1, 1), jnp.float32),
      ],
  )(xt, Wqs, Wk, Wv, WoW1, W2, Wout3,
    Bq, Bk, Bv, b1f, b2.reshape(1, _SW), bout.reshape(1, _EMB))

  n = jnp.float32(B2 * _EMB)
  mu = s1[0, 0] / n
  var = s2[0, 0] / n - mu * mu
  inv = 1.0 / jnp.sqrt(var + 1e-8)
  mu = mu.reshape(1, 1)
  inv = inv.reshape(1, 1)

  pe_big = _sinusoidal(T, _EMB)
  o_col = o_enc.astype(jnp.float32).reshape(B, T, 1)

  out = pl.pallas_call(
      _stage_b,
      grid=(B,),
      in_specs=[
          pl.BlockSpec((1, T, _EMB), lambda i: (i, 0, 0)),
          pl.BlockSpec((1, T, _EMB), lambda i: (i, 0, 0)),
          pl.BlockSpec((1, T, 1), lambda i: (i, 0, 0)),
          pl.BlockSpec((T, _EMB), lambda i: (0, 0)),
          pl.BlockSpec((1, 1), lambda i: (0, 0)),
          pl.BlockSpec((1, 1), lambda i: (0, 0)),
      ],
      out_specs=pl.BlockSpec((1, T, _EMB), lambda i: (i, 0, 0)),
      out_shape=jax.ShapeDtypeStruct((B, T, _EMB), jnp.float32),
  )(emb.reshape(B, T, _EMB), r_enc, o_col, pe_big, mu, inv)
  return out


# speedup vs baseline: 1.2665x; 1.1312x over previous
import functools
import math

import jax
import jax.numpy as jnp
from jax import lax
from jax.experimental import pallas as pl
from jax.experimental.pallas import tpu as pltpu

_SW = 32
_FB = 16
_EMB = 256
_TOK = 512
_G = _TOK * _FB // 128
_BLK = 128


def _sinusoidal(L, D):
  pos = jnp.arange(L, dtype=jnp.float32)[:, None]
  idx = jnp.arange(D, dtype=jnp.float32)[None, :]
  angle = pos / jnp.power(10000.0, 2.0 * jnp.floor(idx / 2.0) / D)
  return jnp.where(jnp.arange(D)[None, :] % 2 == 0, jnp.sin(angle),
                   jnp.cos(angle))


def _stage_a(xt_ref, wq_ref, wk_ref, wv_ref, w1_ref, w2_ref,
             wout_ref, bq_ref, bk_ref, bv_ref, b1_ref, b2_ref,
             bout_ref, emb_ref, s1_ref, s2_ref):
  step = pl.program_id(0)
  x3 = xt_ref[...]
  def mm(a, w):
    return lax.dot_general(a, w, (((2,), (0,)), ((), ())),
                           preferred_element_type=jnp.float32)
  q = mm(x3, wq_ref[...]) + bq_ref[...][None]
  k = mm(x3, wk_ref[...]) + bk_ref[...][None]
  v = mm(x3, wv_ref[...]) + bv_ref[...][None]

  bf = jnp.bfloat16
  s = lax.dot_general(q.astype(bf), k.astype(bf), (((2,), (2,)), ((0,), (0,))),
                      preferred_element_type=jnp.float32)
  i = lax.broadcasted_iota(jnp.int32, (128, 128), 0)
  j = lax.broadcasted_iota(jnp.int32, (128, 128), 1)
  mask = jnp.where((i // _FB) == (j // _FB), 0.0, -1e30).astype(jnp.float32)
  s = s + mask
  m = jnp.max(s, axis=-1, keepdims=True)
  e = jnp.exp(s - m)
  o = lax.dot_general(e.astype(bf), v.astype(bf), (((2,), (1,)), ((0,), (0,))),
                      preferred_element_type=jnp.float32)
  o = o / jnp.sum(e, axis=-1, keepdims=True)
  h = jnp.maximum(mm(o.astype(bf), w1_ref[...]) + b1_ref[...][None], 0.0)
  e2 = mm(h.astype(bf), w2_ref[...]) + b2_ref[...][None]
  e3 = e2.reshape(_TOK, _FB, _SW)
  ecat = jnp.concatenate([e3[:, f, :] for f in range(_FB)], axis=1)
  emb = lax.dot_general(ecat.astype(bf), wout_ref[...],
                        (((1,), (0,)), ((), ())),
                        preferred_element_type=jnp.float32) + bout_ref[...]
  emb_ref[...] = emb
  ssum = jnp.sum(emb).reshape(1, 1)
  ssq = jnp.sum(emb * emb).reshape(1, 1)

  @pl.when(step == 0)
  def _():
    s1_ref[...] = ssum
    s2_ref[...] = ssq

  @pl.when(step != 0)
  def _():
    s1_ref[...] += ssum
    s2_ref[...] += ssq


def _stage_b(emb_ref, r_ref, oc_ref, pe_ref, mu_ref, inv_ref, out_ref):
  T = emb_ref.shape[1]
  nblk = T // _BLK
  mu = mu_ref[0, 0]
  inv = inv_ref[0, 0]

  ti = lax.broadcasted_iota(jnp.int32, (_BLK, _BLK), 0)
  tj = lax.broadcasted_iota(jnp.int32, (_BLK, _BLK), 1)
  tri_low = jnp.where(tj <= ti, 1.0, 0.0).astype(jnp.float32)
  upper = (tj >= ti)

  def body(it, carry):
    jb = nblk - 1 - it
    cs, cc = carry
    ob = oc_ref[0, pl.ds(jb * _BLK, _BLK), :]
    cum = tri_low @ ob
    same = cum == cum.reshape(1, _BLK)
    msk = jnp.where(same & upper, 1.0, 0.0).astype(jnp.float32)
    eb = (emb_ref[0, pl.ds(jb * _BLK, _BLK), :] - mu) * inv
    sblk = msk @ eb
    cblk = jnp.sum(msk, axis=1, keepdims=True)
    nb = jnp.where(jb < nblk - 1, jb + 1, nblk - 1)
    onext = oc_ref[0, pl.ds(nb * _BLK, 1), :]
    cont = jnp.where((jb < nblk - 1) & (onext[0, 0] == 0.0), 1.0, 0.0)
    trail = jnp.where(cum == cum[_BLK - 1, 0], cont, 0.0)
    sblk = sblk + trail * cs
    cblk = cblk + trail * cc
    gidx = jb * _BLK + lax.broadcasted_iota(jnp.int32, (_BLK, 1), 0)
    first = (ob > 0.0) | (gidx == 0)
    bm = jnp.where(first, sblk / cblk, 0.0)
    out_ref[0, pl.ds(jb * _BLK, _BLK), :] = (
        r_ref[0, pl.ds(jb * _BLK, _BLK), :]
        + pe_ref[pl.ds(jb * _BLK, _BLK), :] + eb + bm)
    return (sblk[0:1, :], cblk[0:1, :])

  lax.fori_loop(0, nblk, body,
                (jnp.zeros((1, _EMB), jnp.float32),
                 jnp.zeros((1, 1), jnp.float32)))


@jax.jit
def kernel(x, o_enc, r_enc, Wq, bq, Wk, bk, Wv, bv, Wo, bo, W1, b1, W2, b2,
           Wout, bout):
  B, T, F = x.shape
  B2 = B * T
  xt = jnp.transpose(x.reshape(B2, _SW, _FB), (0, 2, 1)).reshape(
      B2 * _FB // 128, 128, _SW).astype(jnp.bfloat16)
  pe_eff = _sinusoidal(_FB, _SW) * 0.01 + 0.01
  rs = 1.0 / math.sqrt(_SW)
  Wqs = (Wq * rs).astype(jnp.bfloat16)
  Bq = jnp.tile((pe_eff @ Wq + bq) * rs, (128 // _FB, 1))
  Bk = jnp.tile(pe_eff @ Wk + bk, (128 // _FB, 1))
  Bv = jnp.tile(pe_eff @ Wv + bv, (128 // _FB, 1))
  WoW1 = (Wo @ W1).astype(jnp.bfloat16)
  b1f = (bo @ W1 + b1).reshape(1, _SW * 4)
  Wout3 = Wout.reshape(_SW, _FB, _EMB).transpose(1, 0, 2).reshape(
      F, _EMB).astype(jnp.bfloat16)
  Wkb = Wk.astype(jnp.bfloat16)
  Wvb = Wv.astype(jnp.bfloat16)
  W2b = W2.astype(jnp.bfloat16)

  grid_a = B2 // _TOK
  emb, s1, s2 = pl.pallas_call(
      _stage_a,
      grid=(grid_a,),
      in_specs=[
          pl.BlockSpec((_G, 128, _SW), lambda i: (i, 0, 0)),
          *[pl.BlockSpec(w.shape, lambda i: (0,) * w.ndim) for w in
            (Wqs, Wkb, Wvb, WoW1, W2b)],
          pl.BlockSpec((F, _EMB), lambda i: (0, 0)),
          pl.BlockSpec((128, _SW), lambda i: (0, 0)),
          pl.BlockSpec((128, _SW), lambda i: (0, 0)),
          pl.BlockSpec((128, _SW), lambda i: (0, 0)),
          pl.BlockSpec((1, _SW * 4), lambda i: (0, 0)),
          pl.BlockSpec((1, _SW), lambda i: (0, 0)),
          pl.BlockSpec((1, _EMB), lambda i: (0, 0)),
      ],
      out_specs=[
          pl.BlockSpec((_TOK, _EMB), lambda i: (i, 0)),
          pl.BlockSpec((1, 1), lambda i: (0, 0)),
          pl.BlockSpec((1, 1), lambda i: (0, 0)),
      ],
      out_shape=[
          jax.ShapeDtypeStruct((B2, _EMB), jnp.float32),
          jax.ShapeDtypeStruct((1, 1), jnp.float32),
          jax.ShapeDtypeStruct((1, 1), jnp.float32),
      ],
  )(xt, Wqs, Wkb, Wvb, WoW1, W2b, Wout3,
    Bq, Bk, Bv, b1f, b2.reshape(1, _SW), bout.reshape(1, _EMB))

  n = jnp.float32(B2 * _EMB)
  mu = s1[0, 0] / n
  var = s2[0, 0] / n - mu * mu
  inv = 1.0 / jnp.sqrt(var + 1e-8)
  mu = mu.reshape(1, 1)
  inv = inv.reshape(1, 1)

  pe_big = _sinusoidal(T, _EMB)
  o_col = o_enc.astype(jnp.float32).reshape(B, T, 1)

  out = pl.pallas_call(
      _stage_b,
      grid=(B,),
      in_specs=[
          pl.BlockSpec((1, T, _EMB), lambda i: (i, 0, 0)),
          pl.BlockSpec((1, T, _EMB), lambda i: (i, 0, 0)),
          pl.BlockSpec((1, T, 1), lambda i: (i, 0, 0)),
          pl.BlockSpec((T, _EMB), lambda i: (0, 0)),
          pl.BlockSpec((1, 1), lambda i: (0, 0)),
          pl.BlockSpec((1, 1), lambda i: (0, 0)),
      ],
      out_specs=pl.BlockSpec((1, T, _EMB), lambda i: (i, 0, 0)),
      out_shape=jax.ShapeDtypeStruct((B, T, _EMB), jnp.float32),
  )(emb.reshape(B, T, _EMB), r_enc, o_col, pe_big, mu, inv)
  return out


# speedup vs baseline: 1.2690x; 1.0020x over previous
import functools
import math

import jax
import jax.numpy as jnp
from jax import lax
from jax.experimental import pallas as pl
from jax.experimental.pallas import tpu as pltpu

_SW = 32
_FB = 16
_EMB = 256
_TOK = 512
_G = _TOK * _FB // 128
_BLK = 128


def _sinusoidal(L, D):
  pos = jnp.arange(L, dtype=jnp.float32)[:, None]
  idx = jnp.arange(D, dtype=jnp.float32)[None, :]
  angle = pos / jnp.power(10000.0, 2.0 * jnp.floor(idx / 2.0) / D)
  return jnp.where(jnp.arange(D)[None, :] % 2 == 0, jnp.sin(angle),
                   jnp.cos(angle))


def _stage_a(xt_ref, wq_ref, wk_ref, wv_ref, w1_ref, w2_ref,
             wout_ref, bq_ref, bk_ref, bv_ref, b1_ref, b2_ref,
             bout_ref, emb_ref, s1_ref, s2_ref):
  step = pl.program_id(0)
  x3 = xt_ref[...]
  def mm(a, w):
    return lax.dot_general(a, w, (((2,), (0,)), ((), ())),
                           preferred_element_type=jnp.float32)
  q = mm(x3, wq_ref[...]) + bq_ref[...][None]
  k = mm(x3, wk_ref[...]) + bk_ref[...][None]
  v = mm(x3, wv_ref[...]) + bv_ref[...][None]

  bf = jnp.bfloat16
  s = lax.dot_general(q.astype(bf), k.astype(bf), (((2,), (2,)), ((0,), (0,))),
                      preferred_element_type=jnp.float32)
  i = lax.broadcasted_iota(jnp.int32, (128, 128), 0)
  j = lax.broadcasted_iota(jnp.int32, (128, 128), 1)
  mask = jnp.where((i // _FB) == (j // _FB), 0.0, -1e30).astype(jnp.float32)
  s = s + mask
  m = jnp.max(s, axis=-1, keepdims=True)
  e = jnp.exp(s - m)
  o = lax.dot_general(e.astype(bf), v.astype(bf), (((2,), (1,)), ((0,), (0,))),
                      preferred_element_type=jnp.float32)
  o = o / jnp.sum(e, axis=-1, keepdims=True)
  h = jnp.maximum(mm(o.astype(bf), w1_ref[...]) + b1_ref[...][None], 0.0)
  e2 = mm(h.astype(bf), w2_ref[...]) + b2_ref[...][None]
  e3 = e2.reshape(_TOK, _FB, _SW)
  ecat = jnp.concatenate([e3[:, f, :] for f in range(_FB)], axis=1)
  emb = lax.dot_general(ecat.astype(bf), wout_ref[...],
                        (((1,), (0,)), ((), ())),
                        preferred_element_type=jnp.float32) + bout_ref[...]
  emb_ref[...] = emb.astype(bf)
  ssum = jnp.sum(emb).reshape(1, 1)
  ssq = jnp.sum(emb * emb).reshape(1, 1)

  @pl.when(step == 0)
  def _():
    s1_ref[...] = ssum
    s2_ref[...] = ssq

  @pl.when(step != 0)
  def _():
    s1_ref[...] += ssum
    s2_ref[...] += ssq


def _stage_b(emb_ref, r_ref, oc_ref, pe_ref, mu_ref, inv_ref, out_ref):
  T = emb_ref.shape[1]
  nblk = T // _BLK
  mu = mu_ref[0, 0]
  inv = inv_ref[0, 0]

  ti = lax.broadcasted_iota(jnp.int32, (_BLK, _BLK), 0)
  tj = lax.broadcasted_iota(jnp.int32, (_BLK, _BLK), 1)
  tri_low = jnp.where(tj <= ti, 1.0, 0.0).astype(jnp.float32)
  upper = (tj >= ti)

  def body(it, carry):
    jb = nblk - 1 - it
    cs, cc = carry
    ob = oc_ref[0, pl.ds(jb * _BLK, _BLK), :]
    cum = tri_low @ ob
    same = cum == cum.reshape(1, _BLK)
    msk = jnp.where(same & upper, 1.0, 0.0).astype(jnp.float32)
    eb = (emb_ref[0, pl.ds(jb * _BLK, _BLK), :].astype(jnp.float32)
          - mu) * inv
    sblk = msk @ eb
    cblk = jnp.sum(msk, axis=1, keepdims=True)
    nb = jnp.where(jb < nblk - 1, jb + 1, nblk - 1)
    onext = oc_ref[0, pl.ds(nb * _BLK, 1), :]
    cont = jnp.where((jb < nblk - 1) & (onext[0, 0] == 0.0), 1.0, 0.0)
    trail = jnp.where(cum == cum[_BLK - 1, 0], cont, 0.0)
    sblk = sblk + trail * cs
    cblk = cblk + trail * cc
    gidx = jb * _BLK + lax.broadcasted_iota(jnp.int32, (_BLK, 1), 0)
    first = (ob > 0.0) | (gidx == 0)
    bm = jnp.where(first, sblk / cblk, 0.0)
    out_ref[0, pl.ds(jb * _BLK, _BLK), :] = (
        r_ref[0, pl.ds(jb * _BLK, _BLK), :]
        + pe_ref[pl.ds(jb * _BLK, _BLK), :] + eb + bm)
    return (sblk[0:1, :], cblk[0:1, :])

  lax.fori_loop(0, nblk, body,
                (jnp.zeros((1, _EMB), jnp.float32),
                 jnp.zeros((1, 1), jnp.float32)))


@jax.jit
def kernel(x, o_enc, r_enc, Wq, bq, Wk, bk, Wv, bv, Wo, bo, W1, b1, W2, b2,
           Wout, bout):
  B, T, F = x.shape
  B2 = B * T
  xt = jnp.transpose(x.reshape(B2, _SW, _FB), (0, 2, 1)).reshape(
      B2 * _FB // 128, 128, _SW).astype(jnp.bfloat16)
  pe_eff = _sinusoidal(_FB, _SW) * 0.01 + 0.01
  rs = 1.0 / math.sqrt(_SW)
  Wqs = (Wq * rs).astype(jnp.bfloat16)
  Bq = jnp.tile((pe_eff @ Wq + bq) * rs, (128 // _FB, 1))
  Bk = jnp.tile(pe_eff @ Wk + bk, (128 // _FB, 1))
  Bv = jnp.tile(pe_eff @ Wv + bv, (128 // _FB, 1))
  WoW1 = (Wo @ W1).astype(jnp.bfloat16)
  b1f = (bo @ W1 + b1).reshape(1, _SW * 4)
  Wout3 = Wout.reshape(_SW, _FB, _EMB).transpose(1, 0, 2).reshape(
      F, _EMB).astype(jnp.bfloat16)
  Wkb = Wk.astype(jnp.bfloat16)
  Wvb = Wv.astype(jnp.bfloat16)
  W2b = W2.astype(jnp.bfloat16)

  grid_a = B2 // _TOK
  emb, s1, s2 = pl.pallas_call(
      _stage_a,
      grid=(grid_a,),
      in_specs=[
          pl.BlockSpec((_G, 128, _SW), lambda i: (i, 0, 0)),
          *[pl.BlockSpec(w.shape, lambda i: (0,) * w.ndim) for w in
            (Wqs, Wkb, Wvb, WoW1, W2b)],
          pl.BlockSpec((F, _EMB), lambda i: (0, 0)),
          pl.BlockSpec((128, _SW), lambda i: (0, 0)),
          pl.BlockSpec((128, _SW), lambda i: (0, 0)),
          pl.BlockSpec((128, _SW), lambda i: (0, 0)),
          pl.BlockSpec((1, _SW * 4), lambda i: (0, 0)),
          pl.BlockSpec((1, _SW), lambda i: (0, 0)),
          pl.BlockSpec((1, _EMB), lambda i: (0, 0)),
      ],
      out_specs=[
          pl.BlockSpec((_TOK, _EMB), lambda i: (i, 0)),
          pl.BlockSpec((1, 1), lambda i: (0, 0)),
          pl.BlockSpec((1, 1), lambda i: (0, 0)),
      ],
      out_shape=[
          jax.ShapeDtypeStruct((B2, _EMB), jnp.bfloat16),
          jax.ShapeDtypeStruct((1, 1), jnp.float32),
          jax.ShapeDtypeStruct((1, 1), jnp.float32),
      ],
  )(xt, Wqs, Wkb, Wvb, WoW1, W2b, Wout3,
    Bq, Bk, Bv, b1f, b2.reshape(1, _SW), bout.reshape(1, _EMB))

  n = jnp.float32(B2 * _EMB)
  mu = s1[0, 0] / n
  var = s2[0, 0] / n - mu * mu
  inv = 1.0 / jnp.sqrt(var + 1e-8)
  mu = mu.reshape(1, 1)
  inv = inv.reshape(1, 1)

  pe_big = _sinusoidal(T, _EMB)
  o_col = o_enc.astype(jnp.float32).reshape(B, T, 1)

  out = pl.pallas_call(
      _stage_b,
      grid=(B,),
      in_specs=[
          pl.BlockSpec((1, T, _EMB), lambda i: (i, 0, 0)),
          pl.BlockSpec((1, T, _EMB), lambda i: (i, 0, 0)),
          pl.BlockSpec((1, T, 1), lambda i: (i, 0, 0)),
          pl.BlockSpec((T, _EMB), lambda i: (0, 0)),
          pl.BlockSpec((1, 1), lambda i: (0, 0)),
          pl.BlockSpec((1, 1), lambda i: (0, 0)),
      ],
      out_specs=pl.BlockSpec((1, T, _EMB), lambda i: (i, 0, 0)),
      out_shape=jax.ShapeDtypeStruct((B, T, _EMB), jnp.float32),
  )(emb.reshape(B, T, _EMB), r_enc, o_col, pe_big, mu, inv)
  return out


# speedup vs baseline: 1.2698x; 1.0006x over previous
import math

import jax
import jax.numpy as jnp
from jax import lax
from jax.experimental import pallas as pl

_SW = 32
_FB = 16
_EMB = 256
_TOK = 512
_G = _TOK * _FB // 128
_BLK = 128


def _sinusoidal(L, D):
  pos = jnp.arange(L, dtype=jnp.float32)[:, None]
  idx = jnp.arange(D, dtype=jnp.float32)[None, :]
  angle = pos / jnp.power(10000.0, 2.0 * jnp.floor(idx / 2.0) / D)
  return jnp.where(jnp.arange(D)[None, :] % 2 == 0, jnp.sin(angle),
                   jnp.cos(angle))


def _stage_a(xt_ref, wq_ref, wk_ref, wv_ref, w1_ref, w2_ref,
             wout_ref, bq_ref, bk_ref, bv_ref, b1_ref, b2_ref,
             bout_ref, emb_ref, s1_ref, s2_ref):
  step = pl.program_id(0)
  x3 = xt_ref[...]
  def mm(a, w):
    return lax.dot_general(a, w, (((2,), (0,)), ((), ())),
                           preferred_element_type=jnp.float32)
  q = mm(x3, wq_ref[...]) + bq_ref[...][None]
  k = mm(x3, wk_ref[...]) + bk_ref[...][None]
  v = mm(x3, wv_ref[...]) + bv_ref[...][None]

  bf = jnp.bfloat16
  s = lax.dot_general(q.astype(bf), k.astype(bf), (((2,), (2,)), ((0,), (0,))),
                      preferred_element_type=jnp.float32)
  i = lax.broadcasted_iota(jnp.int32, (128, 128), 0)
  j = lax.broadcasted_iota(jnp.int32, (128, 128), 1)
  mask = jnp.where((i // _FB) == (j // _FB), 0.0, -1e30).astype(jnp.float32)
  s = s + mask
  m = jnp.max(s, axis=-1, keepdims=True)
  e = jnp.exp(s - m)
  o = lax.dot_general(e.astype(bf), v.astype(bf), (((2,), (1,)), ((0,), (0,))),
                      preferred_element_type=jnp.float32)
  o = o / jnp.sum(e, axis=-1, keepdims=True)
  h = jnp.maximum(mm(o.astype(bf), w1_ref[...]) + b1_ref[...][None], 0.0)
  e2 = mm(h.astype(bf), w2_ref[...]) + b2_ref[...][None]
  e3 = e2.reshape(_TOK, _FB, _SW)
  ecat = jnp.concatenate([e3[:, f, :] for f in range(_FB)], axis=1)
  emb = lax.dot_general(ecat.astype(bf), wout_ref[...],
                        (((1,), (0,)), ((), ())),
                        preferred_element_type=jnp.float32) + bout_ref[...]
  emb_ref[...] = emb.astype(bf)
  ssum = jnp.sum(emb).reshape(1, 1)
  ssq = jnp.sum(emb * emb).reshape(1, 1)

  @pl.when(step == 0)
  def _():
    s1_ref[...] = ssum
    s2_ref[...] = ssq

  @pl.when(step != 0)
  def _():
    s1_ref[...] += ssum
    s2_ref[...] += ssq


def _stage_b(emb_ref, r_ref, oc_ref, pe_ref, mu_ref, inv_ref, out_ref):
  T = emb_ref.shape[1]
  nblk = T // _BLK
  mu = mu_ref[0, 0]
  inv = inv_ref[0, 0]

  ti = lax.broadcasted_iota(jnp.int32, (_BLK, _BLK), 0)
  tj = lax.broadcasted_iota(jnp.int32, (_BLK, _BLK), 1)
  tri_low = jnp.where(tj <= ti, 1.0, 0.0).astype(jnp.float32)
  upper = (tj >= ti)

  def body(it, carry):
    jb = nblk - 1 - it
    cs, cc = carry
    ob = oc_ref[0, pl.ds(jb * _BLK, _BLK), :]
    cum = tri_low @ ob
    same = cum == cum.reshape(1, _BLK)
    msk = jnp.where(same & upper, 1.0, 0.0).astype(jnp.float32)
    eb = (emb_ref[0, pl.ds(jb * _BLK, _BLK), :].astype(jnp.float32)
          - mu) * inv
    sblk = msk @ eb
    cblk = jnp.sum(msk, axis=1, keepdims=True)
    nb = jnp.where(jb < nblk - 1, jb + 1, nblk - 1)
    onext = oc_ref[0, pl.ds(nb * _BLK, 1), :]
    cont = jnp.where((jb < nblk - 1) & (onext[0, 0] == 0.0), 1.0, 0.0)
    trail = jnp.where(cum == cum[_BLK - 1, 0], cont, 0.0)
    sblk = sblk + trail * cs
    cblk = cblk + trail * cc
    gidx = jb * _BLK + lax.broadcasted_iota(jnp.int32, (_BLK, 1), 0)
    first = (ob > 0.0) | (gidx == 0)
    bm = jnp.where(first, sblk / cblk, 0.0)
    out_ref[0, pl.ds(jb * _BLK, _BLK), :] = (
        r_ref[0, pl.ds(jb * _BLK, _BLK), :]
        + pe_ref[pl.ds(jb * _BLK, _BLK), :] + eb + bm)
    return (sblk[0:1, :], cblk[0:1, :])

  lax.fori_loop(0, nblk, body,
                (jnp.zeros((1, _EMB), jnp.float32),
                 jnp.zeros((1, 1), jnp.float32)))


@jax.jit
def kernel(x, o_enc, r_enc, Wq, bq, Wk, bk, Wv, bv, Wo, bo, W1, b1, W2, b2,
           Wout, bout):
  B, T, F = x.shape
  B2 = B * T
  xt = jnp.transpose(x.reshape(B2, _SW, _FB), (0, 2, 1)).reshape(
      B2 * _FB // 128, 128, _SW).astype(jnp.bfloat16)
  pe_eff = _sinusoidal(_FB, _SW) * 0.01 + 0.01
  rs = 1.0 / math.sqrt(_SW)
  Wqs = (Wq * rs).astype(jnp.bfloat16)
  Bq = jnp.tile((pe_eff @ Wq + bq) * rs, (128 // _FB, 1))
  Bk = jnp.tile(pe_eff @ Wk + bk, (128 // _FB, 1))
  Bv = jnp.tile(pe_eff @ Wv + bv, (128 // _FB, 1))
  WoW1 = (Wo @ W1).astype(jnp.bfloat16)
  b1f = (bo @ W1 + b1).reshape(1, _SW * 4)
  Wout3 = Wout.reshape(_SW, _FB, _EMB).transpose(1, 0, 2).reshape(
      F, _EMB).astype(jnp.bfloat16)
  Wkb = Wk.astype(jnp.bfloat16)
  Wvb = Wv.astype(jnp.bfloat16)
  W2b = W2.astype(jnp.bfloat16)

  grid_a = B2 // _TOK
  emb, s1, s2 = pl.pallas_call(
      _stage_a,
      grid=(grid_a,),
      in_specs=[
          pl.BlockSpec((_G, 128, _SW), lambda i: (i, 0, 0)),
          *[pl.BlockSpec(w.shape, lambda i: (0,) * w.ndim) for w in
            (Wqs, Wkb, Wvb, WoW1, W2b)],
          pl.BlockSpec((F, _EMB), lambda i: (0, 0)),
          pl.BlockSpec((128, _SW), lambda i: (0, 0)),
          pl.BlockSpec((128, _SW), lambda i: (0, 0)),
          pl.BlockSpec((128, _SW), lambda i: (0, 0)),
          pl.BlockSpec((1, _SW * 4), lambda i: (0, 0)),
          pl.BlockSpec((1, _SW), lambda i: (0, 0)),
          pl.BlockSpec((1, _EMB), lambda i: (0, 0)),
      ],
      out_specs=[
          pl.BlockSpec((_TOK, _EMB), lambda i: (i, 0)),
          pl.BlockSpec((1, 1), lambda i: (0, 0)),
          pl.BlockSpec((1, 1), lambda i: (0, 0)),
      ],
      out_shape=[
          jax.ShapeDtypeStruct((B2, _EMB), jnp.bfloat16),
          jax.ShapeDtypeStruct((1, 1), jnp.float32),
          jax.ShapeDtypeStruct((1, 1), jnp.float32),
      ],
  )(xt, Wqs, Wkb, Wvb, WoW1, W2b, Wout3,
    Bq, Bk, Bv, b1f, b2.reshape(1, _SW), bout.reshape(1, _EMB))

  n = jnp.float32(B2 * _EMB)
  mu = s1[0, 0] / n
  var = s2[0, 0] / n - mu * mu
  inv = 1.0 / jnp.sqrt(var + 1e-8)
  mu = mu.reshape(1, 1)
  inv = inv.reshape(1, 1)

  pe_big = _sinusoidal(T, _EMB)
  o_col = o_enc.astype(jnp.float32).reshape(B, T, 1)

  out = pl.pallas_call(
      _stage_b,
      grid=(B,),
      in_specs=[
          pl.BlockSpec((1, T, _EMB), lambda i: (i, 0, 0)),
          pl.BlockSpec((1, T, _EMB), lambda i: (i, 0, 0)),
          pl.BlockSpec((1, T, 1), lambda i: (i, 0, 0)),
          pl.BlockSpec((T, _EMB), lambda i: (0, 0)),
          pl.BlockSpec((1, 1), lambda i: (0, 0)),
          pl.BlockSpec((1, 1), lambda i: (0, 0)),
      ],
      out_specs=pl.BlockSpec((1, T, _EMB), lambda i: (i, 0, 0)),
      out_shape=jax.ShapeDtypeStruct((B, T, _EMB), jnp.float32),
  )(emb.reshape(B, T, _EMB), r_enc, o_col, pe_big, mu, inv)
  return out
